# baseline (speedup 1.0000x reference)
import functools

import jax
import jax.numpy as jnp
from jax import lax
from jax.experimental import pallas as pl
from jax.experimental.pallas import tpu as pltpu
from jax.experimental.pallas import tpu_sc as plsc

B, S, D = 4, 8192, 1024
NC, NS, L = 2, 16, 16
NW = NC * NS
ROWS_W = S // NW
CH = 8
N_CHUNKS = ROWS_W // CH
N_PAIRS = N_CHUNKS // 2

_mesh = plsc.VectorSubcoreMesh(
    core_axis_name="c", subcore_axis_name="s", num_cores=NC, num_subcores=NS
)


def _add_chunk(o_ref, t_ref):
    @plsc.parallel_loop(0, CH, step=1, unroll=2)
    def body(r):
        for j in range(D // L):
            sl = pl.ds(j * L, L)
            plsc.addupdate(o_ref.at[r, sl], t_ref[r, sl])


@functools.partial(
    pl.kernel,
    out_type=jax.ShapeDtypeStruct((B, S, D), jnp.float32),
    mesh=_mesh,
    scratch_types=[
        [pltpu.VMEM((CH, D), jnp.float32)] * 2,
        [pltpu.VMEM((CH, D), jnp.float32)] * 8,
        [pltpu.SemaphoreType.DMA] * 2,
        [pltpu.SemaphoreType.DMA] * 8,
        [pltpu.SemaphoreType.DMA] * 8,
    ],
    compiler_params=pltpu.CompilerParams(use_tc_tiling_on_sc=True),
)
def _pos_add_sc(x_hbm, pos_hbm, out_hbm, t, o, st, si, so):
    wid = lax.axis_index("s") * NC + lax.axis_index("c")
    base = wid * ROWS_W

    def tin(c, k):
        return pltpu.make_async_copy(pos_hbm.at[pl.ds(base + c * CH, CH)], t[k], st[k])

    def xin(b, c, j):
        return pltpu.make_async_copy(x_hbm.at[b, pl.ds(base + c * CH, CH)], o[j], si[j])

    def xout(b, c, j):
        return pltpu.make_async_copy(o[j], out_hbm.at[b, pl.ds(base + c * CH, CH)], so[j])

    def pair(i, _):
        c0 = 2 * i
        c1 = c0 + 1
        tin(c0, 0).wait()
        for j in range(4):
            xin(j, c0, j).wait()
            _add_chunk(o[j], t[0])
            xout(j, c0, j).start()

            @pl.when(c0 >= 1)
            def _():
                xout(j, c1 - 2, j + 4).wait()

            xin(j, c1, j + 4).start()

        @pl.when(c0 + 2 < N_CHUNKS)
        def _():
            tin(c0 + 2, 0).start()

        tin(c1, 1).wait()
        for j in range(4):
            xin(j, c1, j + 4).wait()
            _add_chunk(o[j + 4], t[1])
            xout(j, c1, j + 4).start()

            @pl.when(c0 + 2 < N_CHUNKS)
            def _():
                xout(j, c0, j).wait()
                xin(j, c0 + 2, j).start()

        @pl.when(c1 + 2 < N_CHUNKS)
        def _():
            tin(c1 + 2, 1).start()

        return 0

    tin(0, 0).start()
    tin(1, 1).start()
    for j in range(4):
        xin(j, 0, j).start()

    lax.fori_loop(0, N_PAIRS, pair, 0, unroll=False)

    for j in range(4):
        xout(j, N_CHUNKS - 2, j).wait()
    for j in range(4):
        xout(j, N_CHUNKS - 1, j + 4).wait()


def kernel(x, pos_table):
    return _pos_add_sc(x, pos_table[:S])


# speedup vs baseline: 1.7618x; 1.7618x over previous
import functools

import jax
import jax.numpy as jnp
from jax import lax
from jax.experimental import pallas as pl
from jax.experimental.pallas import tpu as pltpu
from jax.experimental.pallas import tpu_sc as plsc

B, S, D = 4, 8192, 1024
NC, NS, L = 2, 16, 16
NW = NC * NS
ROWS_W = S // NW
CH = 8
N_CHUNKS = ROWS_W // CH
N_PAIRS = N_CHUNKS // 2

_mesh = plsc.VectorSubcoreMesh(
    core_axis_name="c", subcore_axis_name="s", num_cores=NC, num_subcores=NS
)


def _add_chunk(o_ref, t_ref):
    @plsc.parallel_loop(0, CH * (D // L) // 16, step=1, unroll=1)
    def body(g):
        r = g // 4
        col = (g % 4) * (16 * L)
        for j in range(16):
            sl = pl.ds(col + j * L, L)
            plsc.addupdate(o_ref.at[r, sl], t_ref[r, sl])


@functools.partial(
    pl.kernel,
    out_type=jax.ShapeDtypeStruct((B, S, D), jnp.float32),
    mesh=_mesh,
    scratch_types=[
        [pltpu.VMEM((CH, D), jnp.float32)] * 2,
        [pltpu.VMEM((CH, D), jnp.float32)] * 8,
        [pltpu.SemaphoreType.DMA] * 2,
        [pltpu.SemaphoreType.DMA] * 8,
        [pltpu.SemaphoreType.DMA] * 8,
    ],
    compiler_params=pltpu.CompilerParams(use_tc_tiling_on_sc=True),
)
def _pos_add_sc(x_hbm, pos_hbm, out_hbm, t, o, st, si, so):
    wid = lax.axis_index("s") * NC + lax.axis_index("c")
    base = wid * ROWS_W

    def tin(c, k):
        return pltpu.make_async_copy(pos_hbm.at[pl.ds(base + c * CH, CH)], t[k], st[k])

    def xin(b, c, j):
        return pltpu.make_async_copy(x_hbm.at[b, pl.ds(base + c * CH, CH)], o[j], si[j])

    def xout(b, c, j):
        return pltpu.make_async_copy(o[j], out_hbm.at[b, pl.ds(base + c * CH, CH)], so[j])

    def pair(i, _):
        c0 = 2 * i
        c1 = c0 + 1
        tin(c0, 0).wait()
        for j in range(4):
            xin(j, c0, j).wait()
            _add_chunk(o[j], t[0])
            xout(j, c0, j).start()

            @pl.when(c0 >= 1)
            def _():
                xout(j, c1 - 2, j + 4).wait()

            xin(j, c1, j + 4).start()

        @pl.when(c0 + 2 < N_CHUNKS)
        def _():
            tin(c0 + 2, 0).start()

        tin(c1, 1).wait()
        for j in range(4):
            xin(j, c1, j + 4).wait()
            _add_chunk(o[j + 4], t[1])
            xout(j, c1, j + 4).start()

            @pl.when(c0 + 2 < N_CHUNKS)
            def _():
                xout(j, c0, j).wait()
                xin(j, c0 + 2, j).start()

        @pl.when(c1 + 2 < N_CHUNKS)
        def _():
            tin(c1 + 2, 1).start()

        return 0

    tin(0, 0).start()
    tin(1, 1).start()
    for j in range(4):
        xin(j, 0, j).start()

    lax.fori_loop(0, N_PAIRS, pair, 0, unroll=False)

    for j in range(4):
        xout(j, N_CHUNKS - 2, j).wait()
    for j in range(4):
        xout(j, N_CHUNKS - 1, j + 4).wait()


def kernel(x, pos_table):
    return _pos_add_sc(x, pos_table[:S])


# speedup vs baseline: 1.7953x; 1.0191x over previous
import functools

import jax
import jax.numpy as jnp
from jax import lax
from jax.experimental import pallas as pl
from jax.experimental.pallas import tpu as pltpu
from jax.experimental.pallas import tpu_sc as plsc

B, S, D = 4, 8192, 1024
NC, NS, L = 2, 16, 16
NW = NC * NS
ROWS_W = S // NW
CH = 8
N_CHUNKS = ROWS_W // CH
N_PAIRS = N_CHUNKS // 2

_mesh = plsc.VectorSubcoreMesh(
    core_axis_name="c", subcore_axis_name="s", num_cores=NC, num_subcores=NS
)


def _add_chunk(o_ref, t_ref):
    @plsc.parallel_loop(0, CH * (D // L) // 8, step=1, unroll=1)
    def body(g):
        r = g // 8
        col = (g % 8) * (8 * L)
        for j in range(8):
            sl = pl.ds(col + j * L, L)
            plsc.addupdate(o_ref.at[r, sl], t_ref[r, sl])


@functools.partial(
    pl.kernel,
    out_type=jax.ShapeDtypeStruct((B, S, D), jnp.float32),
    mesh=_mesh,
    scratch_types=[
        [pltpu.VMEM((CH, D), jnp.float32)] * 2,
        [pltpu.VMEM((CH, D), jnp.float32)] * 8,
        [pltpu.SemaphoreType.DMA] * 2,
        [pltpu.SemaphoreType.DMA] * 8,
        [pltpu.SemaphoreType.DMA] * 8,
    ],
    compiler_params=pltpu.CompilerParams(use_tc_tiling_on_sc=True),
)
def _pos_add_sc(x_hbm, pos_hbm, out_hbm, t, o, st, si, so):
    wid = lax.axis_index("s") * NC + lax.axis_index("c")
    base = wid * ROWS_W

    def tin(c, k):
        return pltpu.make_async_copy(pos_hbm.at[pl.ds(base + c * CH, CH)], t[k], st[k])

    def xin(b, c, j):
        return pltpu.make_async_copy(x_hbm.at[b, pl.ds(base + c * CH, CH)], o[j], si[j])

    def xout(b, c, j):
        return pltpu.make_async_copy(o[j], out_hbm.at[b, pl.ds(base + c * CH, CH)], so[j])

    def pair(i, _):
        c0 = 2 * i
        c1 = c0 + 1
        tin(c0, 0).wait()
        for j in range(4):
            xin(j, c0, j).wait()
            _add_chunk(o[j], t[0])
            xout(j, c0, j).start()

            @pl.when(c0 >= 1)
            def _():
                xout(j, c1 - 2, j + 4).wait()

            xin(j, c1, j + 4).start()

        @pl.when(c0 + 2 < N_CHUNKS)
        def _():
            tin(c0 + 2, 0).start()

        tin(c1, 1).wait()
        for j in range(4):
            xin(j, c1, j + 4).wait()
            _add_chunk(o[j + 4], t[1])
            xout(j, c1, j + 4).start()

            @pl.when(c0 + 2 < N_CHUNKS)
            def _():
                xout(j, c0, j).wait()
                xin(j, c0 + 2, j).start()

        @pl.when(c1 + 2 < N_CHUNKS)
        def _():
            tin(c1 + 2, 1).start()

        return 0

    tin(0, 0).start()
    tin(1, 1).start()
    for j in range(4):
        xin(j, 0, j).start()

    lax.fori_loop(0, N_PAIRS, pair, 0, unroll=False)

    for j in range(4):
        xout(j, N_CHUNKS - 2, j).wait()
    for j in range(4):
        xout(j, N_CHUNKS - 1, j + 4).wait()


def kernel(x, pos_table):
    return _pos_add_sc(x, pos_table[:S])


# speedup vs baseline: 1.8100x; 1.0082x over previous
import functools

import jax
import jax.numpy as jnp
from jax import lax
from jax.experimental import pallas as pl
from jax.experimental.pallas import tpu as pltpu
from jax.experimental.pallas import tpu_sc as plsc

B, S, D = 4, 8192, 1024
NC, NS, L = 2, 16, 16
NW = NC * NS
ROWS_W = S // NW
CH = 8
N_CHUNKS = ROWS_W // CH
N_PAIRS = N_CHUNKS // 2

_mesh = plsc.VectorSubcoreMesh(
    core_axis_name="c", subcore_axis_name="s", num_cores=NC, num_subcores=NS
)


def _add_chunk(o_ref, t_ref):
    @plsc.parallel_loop(0, CH * (D // L) // 4, step=1, unroll=1)
    def body(g):
        r = g // 16
        col = (g % 16) * (4 * L)
        for j in range(4):
            sl = pl.ds(col + j * L, L)
            plsc.addupdate(o_ref.at[r, sl], t_ref[r, sl])


@functools.partial(
    pl.kernel,
    out_type=jax.ShapeDtypeStruct((B, S, D), jnp.float32),
    mesh=_mesh,
    scratch_types=[
        [pltpu.VMEM((CH, D), jnp.float32)] * 2,
        [pltpu.VMEM((CH, D), jnp.float32)] * 8,
        [pltpu.SemaphoreType.DMA] * 2,
        [pltpu.SemaphoreType.DMA] * 8,
        [pltpu.SemaphoreType.DMA] * 8,
    ],
    compiler_params=pltpu.CompilerParams(use_tc_tiling_on_sc=True),
)
def _pos_add_sc(x_hbm, pos_hbm, out_hbm, t, o, st, si, so):
    wid = lax.axis_index("s") * NC + lax.axis_index("c")
    base = wid * ROWS_W

    def tin(c, k):
        return pltpu.make_async_copy(pos_hbm.at[pl.ds(base + c * CH, CH)], t[k], st[k])

    def xin(b, c, j):
        return pltpu.make_async_copy(x_hbm.at[b, pl.ds(base + c * CH, CH)], o[j], si[j])

    def xout(b, c, j):
        return pltpu.make_async_copy(o[j], out_hbm.at[b, pl.ds(base + c * CH, CH)], so[j])

    def pair(i, _):
        c0 = 2 * i
        c1 = c0 + 1
        tin(c0, 0).wait()
        for j in range(4):
            xin(j, c0, j).wait()
            _add_chunk(o[j], t[0])
            xout(j, c0, j).start()

            @pl.when(c0 >= 1)
            def _():
                xout(j, c1 - 2, j + 4).wait()

            xin(j, c1, j + 4).start()

        @pl.when(c0 + 2 < N_CHUNKS)
        def _():
            tin(c0 + 2, 0).start()

        tin(c1, 1).wait()
        for j in range(4):
            xin(j, c1, j + 4).wait()
            _add_chunk(o[j + 4], t[1])
            xout(j, c1, j + 4).start()

            @pl.when(c0 + 2 < N_CHUNKS)
            def _():
                xout(j, c0, j).wait()
                xin(j, c0 + 2, j).start()

        @pl.when(c1 + 2 < N_CHUNKS)
        def _():
            tin(c1 + 2, 1).start()

        return 0

    tin(0, 0).start()
    tin(1, 1).start()
    for j in range(4):
        xin(j, 0, j).start()

    lax.fori_loop(0, N_PAIRS, pair, 0, unroll=False)

    for j in range(4):
        xout(j, N_CHUNKS - 2, j).wait()
    for j in range(4):
        xout(j, N_CHUNKS - 1, j + 4).wait()


def kernel(x, pos_table):
    return _pos_add_sc(x, pos_table[:S])
